# baseline (speedup 1.0000x reference)
import functools

import jax
import jax.numpy as jnp
from jax import lax
from jax.experimental import pallas as pl
from jax.experimental.pallas import tpu as pltpu
from jax.experimental.pallas import tpu_sc as plsc

BATCH = 16384
EMB = 64
H1 = 128
CHUNK = 128


def _gather3_pairs(uid2d, did2d, bid2d, ut_p, dt_p, bt_p):
    info = plsc.get_sparse_core_info()
    nc, ns = info.num_cores, info.num_subcores
    nw = nc * ns
    rows_per_w = BATCH // nw
    nchunk = rows_per_w // CHUNK

    mesh = plsc.VectorSubcoreMesh(core_axis_name="c", subcore_axis_name="s")

    @functools.partial(
        pl.kernel,
        mesh=mesh,
        compiler_params=pltpu.CompilerParams(use_tc_tiling_on_sc=True),
        out_type=(
            jax.ShapeDtypeStruct((BATCH, 2 * EMB), jnp.float32),
            jax.ShapeDtypeStruct((BATCH, 2 * EMB), jnp.float32),
            jax.ShapeDtypeStruct((BATCH, 2 * EMB), jnp.float32),
        ),
        scratch_types=[
            pltpu.VMEM((nchunk, CHUNK), jnp.int32),
            pltpu.VMEM((nchunk, CHUNK), jnp.int32),
            pltpu.VMEM((nchunk, CHUNK), jnp.int32),
            pltpu.VMEM((rows_per_w, 2 * EMB), jnp.float32),
            pltpu.SemaphoreType.DMA,
        ],
    )
    def gather_kernel(uid_hbm, did_hbm, bid_hbm, ut_hbm, dt_hbm, bt_hbm,
                      uo_hbm, do_hbm, bo_hbm,
                      uidx_v, didx_v, bidx_v, rows_v, sem):
        wid = lax.axis_index("s") * nc + lax.axis_index("c")
        rbase = wid * nchunk
        pltpu.sync_copy(uid_hbm.at[pl.ds(rbase, nchunk)], uidx_v)
        pltpu.sync_copy(did_hbm.at[pl.ds(rbase, nchunk)], didx_v)
        pltpu.sync_copy(bid_hbm.at[pl.ds(rbase, nchunk)], bidx_v)
        for idx_v in (uidx_v, didx_v, bidx_v):
            for c in range(nchunk):
                for k in range(CHUNK // 16):
                    s = pl.ds(k * 16, 16)
                    idx_v[c, s] = lax.shift_right_logical(idx_v[c, s], 1)
        base = wid * rows_per_w
        for idx_v, t_hbm, o_hbm in ((uidx_v, ut_hbm, uo_hbm),
                                    (didx_v, dt_hbm, do_hbm),
                                    (bidx_v, bt_hbm, bo_hbm)):
            copies = [
                pltpu.async_copy(t_hbm.at[idx_v.at[c]],
                                 rows_v.at[pl.ds(c * CHUNK, CHUNK)], sem)
                for c in range(nchunk)
            ]
            for cp in copies:
                cp.wait()
            pltpu.sync_copy(rows_v, o_hbm.at[pl.ds(base, rows_per_w)])

    return gather_kernel(uid2d, did2d, bid2d, ut_p, dt_p, bt_p)


def _mlp_body(u_ref, d_ref, b_ref, um_ref, dm_ref, bm_ref,
              w1_ref, b1_ref, w2_ref, b2_ref, w3_ref, b3_ref, o_ref):
    def pick(pair_ref, m_ref):
        m = m_ref[...]
        return pair_ref[:, 0:EMB] * (1.0 - m) + pair_ref[:, EMB:2 * EMB] * m

    u = pick(u_ref, um_ref)
    d = pick(d_ref, dm_ref)
    b = pick(b_ref, bm_ref)
    h = jnp.dot(u, w1_ref[0:EMB, :], preferred_element_type=jnp.float32)
    h = h + jnp.dot(d, w1_ref[EMB:2 * EMB, :], preferred_element_type=jnp.float32)
    h = h + jnp.dot(b, w1_ref[2 * EMB:3 * EMB, :], preferred_element_type=jnp.float32)
    h = jnp.maximum(h + b1_ref[...], 0.0)
    h = jnp.maximum(jnp.dot(h, w2_ref[...], preferred_element_type=jnp.float32) + b2_ref[...], 0.0)
    o = jnp.dot(h, w3_ref[...], preferred_element_type=jnp.float32) + b3_ref[...]
    o_ref[...] = o


def _mlp(u2, d2, b2_, um, dm, bm, W1, b1, W2, b2, W3, b3):
    bb = 2048
    grid = (BATCH // bb,)
    return pl.pallas_call(
        _mlp_body,
        grid=grid,
        in_specs=[
            pl.BlockSpec((bb, 2 * EMB), lambda i: (i, 0)),
            pl.BlockSpec((bb, 2 * EMB), lambda i: (i, 0)),
            pl.BlockSpec((bb, 2 * EMB), lambda i: (i, 0)),
            pl.BlockSpec((bb, 1), lambda i: (i, 0)),
            pl.BlockSpec((bb, 1), lambda i: (i, 0)),
            pl.BlockSpec((bb, 1), lambda i: (i, 0)),
            pl.BlockSpec((3 * EMB, H1), lambda i: (0, 0)),
            pl.BlockSpec((1, H1), lambda i: (0, 0)),
            pl.BlockSpec((H1, EMB), lambda i: (0, 0)),
            pl.BlockSpec((1, EMB), lambda i: (0, 0)),
            pl.BlockSpec((EMB, 1), lambda i: (0, 0)),
            pl.BlockSpec((1, 1), lambda i: (0, 0)),
        ],
        out_specs=pl.BlockSpec((bb, 1), lambda i: (i, 0)),
        out_shape=jax.ShapeDtypeStruct((BATCH, 1), jnp.float32),
    )(u2, d2, b2_, um, dm, bm, W1, b1.reshape(1, H1), W2, b2.reshape(1, EMB),
      W3, b3.reshape(1, 1))


def kernel(user_ids, device_ids, brand_ids, user_table, device_table, brand_table,
           W1, b1, W2, b2, W3, b3):
    uid = user_ids.astype(jnp.int32)
    did = device_ids.astype(jnp.int32)
    bid = brand_ids.astype(jnp.int32)
    ut_p = user_table.reshape(-1, 2 * EMB)
    dt_p = device_table.reshape(-1, 2 * EMB)
    bt_p = brand_table.reshape(-1, 2 * EMB)
    u2, d2, b2_ = _gather3_pairs(
        uid.reshape(BATCH // CHUNK, CHUNK), did.reshape(BATCH // CHUNK, CHUNK),
        bid.reshape(BATCH // CHUNK, CHUNK), ut_p, dt_p, bt_p)
    um = (uid % 2).astype(jnp.float32).reshape(BATCH, 1)
    dm = (did % 2).astype(jnp.float32).reshape(BATCH, 1)
    bm = (bid % 2).astype(jnp.float32).reshape(BATCH, 1)
    out = _mlp(u2, d2, b2_, um, dm, bm, W1, b1, W2, b2, W3, b3)
    return out.reshape(BATCH)


# speedup vs baseline: 2.1369x; 2.1369x over previous
import functools

import jax
import jax.numpy as jnp
from jax import lax
from jax.experimental import pallas as pl
from jax.experimental.pallas import tpu as pltpu
from jax.experimental.pallas import tpu_sc as plsc

BATCH = 16384
EMB = 64
H1 = 128
CHUNK = 128
LR_BIG = 13
LR_SMALL = 9


def _pack_pairs_body(a_ref, b_ref, o_ref):
    o_ref[:, 0:EMB] = jnp.swapaxes(a_ref[...], 0, 1)
    o_ref[:, EMB:2 * EMB] = jnp.swapaxes(b_ref[...], 0, 1)


def _pack_pairs(tT, lr):
    r = 1 << lr
    n = tT.shape[1]
    nblk = (n + 2 * r - 1) // (2 * r)
    last = (n - 1) // r
    return pl.pallas_call(
        _pack_pairs_body,
        grid=(nblk,),
        in_specs=[
            pl.BlockSpec((EMB, r), lambda i: (0, 2 * i)),
            pl.BlockSpec((EMB, r), lambda i: (0, jnp.minimum(2 * i + 1, last))),
        ],
        out_specs=pl.BlockSpec((r, 2 * EMB), lambda i: (i, 0)),
        out_shape=jax.ShapeDtypeStruct((nblk * r, 2 * EMB), jnp.float32),
    )(tT, tT)


def _gather3_pairs(uid2d, did2d, bid2d, ut_p, dt_p, bt_p):
    info = plsc.get_sparse_core_info()
    nc, ns = info.num_cores, info.num_subcores
    nw = nc * ns
    rows_per_w = BATCH // nw
    nchunk = rows_per_w // CHUNK

    mesh = plsc.VectorSubcoreMesh(core_axis_name="c", subcore_axis_name="s")

    @functools.partial(
        pl.kernel,
        mesh=mesh,
        compiler_params=pltpu.CompilerParams(use_tc_tiling_on_sc=True),
        out_type=(
            jax.ShapeDtypeStruct((BATCH, 2 * EMB), jnp.float32),
            jax.ShapeDtypeStruct((BATCH, 2 * EMB), jnp.float32),
            jax.ShapeDtypeStruct((BATCH, 2 * EMB), jnp.float32),
        ),
        scratch_types=[
            pltpu.VMEM((nchunk, CHUNK), jnp.int32),
            pltpu.VMEM((nchunk, CHUNK), jnp.int32),
            pltpu.VMEM((nchunk, CHUNK), jnp.int32),
            pltpu.VMEM((rows_per_w, 2 * EMB), jnp.float32),
            pltpu.SemaphoreType.DMA,
        ],
    )
    def gather_kernel(uid_hbm, did_hbm, bid_hbm, ut_hbm, dt_hbm, bt_hbm,
                      uo_hbm, do_hbm, bo_hbm,
                      uidx_v, didx_v, bidx_v, rows_v, sem):
        wid = lax.axis_index("s") * nc + lax.axis_index("c")
        rbase = wid * nchunk
        pltpu.sync_copy(uid_hbm.at[pl.ds(rbase, nchunk)], uidx_v)
        pltpu.sync_copy(did_hbm.at[pl.ds(rbase, nchunk)], didx_v)
        pltpu.sync_copy(bid_hbm.at[pl.ds(rbase, nchunk)], bidx_v)
        for idx_v, lr in ((uidx_v, LR_BIG), (didx_v, LR_BIG), (bidx_v, LR_SMALL)):
            for c in range(nchunk):
                for k in range(CHUNK // 16):
                    s = pl.ds(k * 16, 16)
                    v = idx_v[c, s]
                    blk = lax.shift_right_logical(v, lr + 1)
                    off = lax.bitwise_and(v, (1 << lr) - 1)
                    idx_v[c, s] = lax.bitwise_or(
                        lax.shift_left(blk, lr), off)
        base = wid * rows_per_w
        for idx_v, t_hbm, o_hbm in ((uidx_v, ut_hbm, uo_hbm),
                                    (didx_v, dt_hbm, do_hbm),
                                    (bidx_v, bt_hbm, bo_hbm)):
            copies = [
                pltpu.async_copy(t_hbm.at[idx_v.at[c]],
                                 rows_v.at[pl.ds(c * CHUNK, CHUNK)], sem)
                for c in range(nchunk)
            ]
            for cp in copies:
                cp.wait()
            pltpu.sync_copy(rows_v, o_hbm.at[pl.ds(base, rows_per_w)])

    return gather_kernel(uid2d, did2d, bid2d, ut_p, dt_p, bt_p)


def _mlp_body(u_ref, d_ref, b_ref, um_ref, dm_ref, bm_ref,
              w1_ref, b1_ref, w2_ref, b2_ref, w3_ref, b3_ref, o_ref):
    def pick(pair_ref, m_ref):
        m = m_ref[...] > 0.5
        return jnp.where(m, pair_ref[:, EMB:2 * EMB], pair_ref[:, 0:EMB])

    u = pick(u_ref, um_ref)
    d = pick(d_ref, dm_ref)
    b = pick(b_ref, bm_ref)
    h = jnp.dot(u, w1_ref[0:EMB, :], preferred_element_type=jnp.float32)
    h = h + jnp.dot(d, w1_ref[EMB:2 * EMB, :], preferred_element_type=jnp.float32)
    h = h + jnp.dot(b, w1_ref[2 * EMB:3 * EMB, :], preferred_element_type=jnp.float32)
    h = jnp.maximum(h + b1_ref[...], 0.0)
    h = jnp.maximum(jnp.dot(h, w2_ref[...], preferred_element_type=jnp.float32) + b2_ref[...], 0.0)
    o = jnp.dot(h, w3_ref[...], preferred_element_type=jnp.float32) + b3_ref[...]
    o_ref[...] = o


def _mlp(u2, d2, b2_, um, dm, bm, W1, b1, W2, b2, W3, b3):
    bb = 2048
    grid = (BATCH // bb,)
    return pl.pallas_call(
        _mlp_body,
        grid=grid,
        in_specs=[
            pl.BlockSpec((bb, 2 * EMB), lambda i: (i, 0)),
            pl.BlockSpec((bb, 2 * EMB), lambda i: (i, 0)),
            pl.BlockSpec((bb, 2 * EMB), lambda i: (i, 0)),
            pl.BlockSpec((bb, 1), lambda i: (i, 0)),
            pl.BlockSpec((bb, 1), lambda i: (i, 0)),
            pl.BlockSpec((bb, 1), lambda i: (i, 0)),
            pl.BlockSpec((3 * EMB, H1), lambda i: (0, 0)),
            pl.BlockSpec((1, H1), lambda i: (0, 0)),
            pl.BlockSpec((H1, EMB), lambda i: (0, 0)),
            pl.BlockSpec((1, EMB), lambda i: (0, 0)),
            pl.BlockSpec((EMB, 1), lambda i: (0, 0)),
            pl.BlockSpec((1, 1), lambda i: (0, 0)),
        ],
        out_specs=pl.BlockSpec((bb, 1), lambda i: (i, 0)),
        out_shape=jax.ShapeDtypeStruct((BATCH, 1), jnp.float32),
    )(u2, d2, b2_, um, dm, bm, W1, b1.reshape(1, H1), W2, b2.reshape(1, EMB),
      W3, b3.reshape(1, 1))


def kernel(user_ids, device_ids, brand_ids, user_table, device_table, brand_table,
           W1, b1, W2, b2, W3, b3):
    uid = user_ids.astype(jnp.int32)
    did = device_ids.astype(jnp.int32)
    bid = brand_ids.astype(jnp.int32)
    ut_p = _pack_pairs(user_table.T, LR_BIG)
    dt_p = _pack_pairs(device_table.T, LR_BIG)
    bt_p = _pack_pairs(brand_table.T, LR_SMALL)
    u2, d2, b2_ = _gather3_pairs(
        uid.reshape(BATCH // CHUNK, CHUNK), did.reshape(BATCH // CHUNK, CHUNK),
        bid.reshape(BATCH // CHUNK, CHUNK), ut_p, dt_p, bt_p)
    um = ((uid >> LR_BIG) & 1).astype(jnp.float32).reshape(BATCH, 1)
    dm = ((did >> LR_BIG) & 1).astype(jnp.float32).reshape(BATCH, 1)
    bm = ((bid >> LR_SMALL) & 1).astype(jnp.float32).reshape(BATCH, 1)
    out = _mlp(u2, d2, b2_, um, dm, bm, W1, b1, W2, b2, W3, b3)
    return out.reshape(BATCH)
